# Initial kernel scaffold; baseline (speedup 1.0000x reference)
import jax
import jax.numpy as jnp
from jax.experimental import pallas as pl


def kernel(x, hyperplanes):
    raise NotImplementedError("write your pallas kernel here")


# speedup vs baseline: 1.0174x; 1.0174x over previous
import jax
import jax.numpy as jnp
from jax.experimental import pallas as pl

_BLOCK = 2048


def _body(x_ref, h_ref, out_ref):
    p = h_ref.shape[1]
    proj = jax.lax.dot_general(
        h_ref[...], x_ref[...], (((0,), (1,)), ((), ())),
        preferred_element_type=jnp.float32)
    m = jnp.max(proj, axis=0, keepdims=True)
    iota = jax.lax.broadcasted_iota(jnp.int32, (p, 1), 0)
    w = (jnp.int32(1) << (p - 1 - iota)).astype(jnp.float32)
    eq = (proj == m).astype(jnp.float32)
    v = jnp.sum(eq * w, axis=0)
    e = (jax.lax.bitcast_convert_type(v, jnp.int32) >> 23) - 127
    out_ref[...] = (p - 1 - e).astype(jnp.int32)


def kernel(x, hyperplanes):
    t, d = x.shape
    p = hyperplanes.shape[1]
    b = _BLOCK
    out = pl.pallas_call(
        _body,
        grid=(t // b,),
        in_specs=[
            pl.BlockSpec((b, d), lambda i: (i, 0)),
            pl.BlockSpec((d, p), lambda i: (0, 0)),
        ],
        out_specs=pl.BlockSpec((b,), lambda i: (i,)),
        out_shape=jax.ShapeDtypeStruct((t,), jnp.int32),
    )(x, hyperplanes)
    return out


# speedup vs baseline: 1.0388x; 1.0211x over previous
import jax
import jax.numpy as jnp
from jax.experimental import pallas as pl

_BLOCK = 4096


def _body(x_ref, h_ref, out_ref):
    p = h_ref.shape[1]
    proj = jax.lax.dot_general(
        h_ref[...], x_ref[...], (((0,), (1,)), ((), ())),
        preferred_element_type=jnp.float32)
    m = jnp.max(proj, axis=0, keepdims=True)
    iota = jax.lax.broadcasted_iota(jnp.int32, (p, 1), 0)
    w = (jnp.int32(1) << (p - 1 - iota)).astype(jnp.float32)
    eq = (proj == m).astype(jnp.float32)
    v = jnp.sum(eq * w, axis=0)
    e = (jax.lax.bitcast_convert_type(v, jnp.int32) >> 23) - 127
    out_ref[...] = (p - 1 - e).astype(jnp.int32)


def kernel(x, hyperplanes):
    t, d = x.shape
    p = hyperplanes.shape[1]
    b = _BLOCK
    out = pl.pallas_call(
        _body,
        grid=(t // b,),
        in_specs=[
            pl.BlockSpec((b, d), lambda i: (i, 0)),
            pl.BlockSpec((d, p), lambda i: (0, 0)),
        ],
        out_specs=pl.BlockSpec((b,), lambda i: (i,)),
        out_shape=jax.ShapeDtypeStruct((t,), jnp.int32),
    )(x, hyperplanes)
    return out
